# baseline (speedup 1.0000x reference)
import functools

import jax
import jax.numpy as jnp
from jax import lax
from jax.experimental import pallas as pl
from jax.experimental.pallas import tpu as pltpu
from jax.experimental.pallas import tpu_sc as plsc

N = 10000
E = 160000
H = 128
D_EDGE = 16

NCORE = 2
NSUB = 16
NW = NCORE * NSUB

CH = 128
NCHUNK = E // CH
CPW = -(-NCHUNK // NW)
ZC = 80
NZCHUNK = N // ZC
ZPW = -(-NZCHUNK // NSUB)

NB = 1000
EB = 3200

_LANES = H // 16

_sc_mesh = lambda: plsc.VectorSubcoreMesh(
    core_axis_name="c", subcore_axis_name="s", num_cores=NCORE, num_subcores=NSUB)


def _sc_gather_body(s_hbm, t_hbm, rc_hbm, g_hbm,
                    rc0, rc1, sb0, tb0, sb1, tb1,
                    semi0, semi1, semg0, semg1, semo0, semo1):
  wid = lax.axis_index("s") * NCORE + lax.axis_index("c")
  rc = (rc0, rc1)
  sb = (sb0, sb1)
  tb = (tb0, tb1)
  semi = (semi0, semi1)
  semg = (semg0, semg1)
  semo = (semo0, semo1)

  def chunk_of(k):
    return wid + k * NW

  def idx_copy(k, b):
    return pltpu.make_async_copy(rc_hbm.at[chunk_of(k)], rc[b], semi[b])

  def gather_copies(k, b):
    return (pltpu.make_async_copy(s_hbm.at[rc[b].at[0]], sb[b], semg[b]),
            pltpu.make_async_copy(t_hbm.at[rc[b].at[1]], tb[b], semg[b]))

  def store_copy(k, b):
    return pltpu.make_async_copy(sb[b], g_hbm.at[pl.ds(chunk_of(k) * CH, CH)],
                                 semo[b])

  def issue_idx(k, b):
    @pl.when(chunk_of(k) < NCHUNK)
    def _():
      idx_copy(k, b).start()

  def issue_gather(k, b):
    @pl.when(chunk_of(k) < NCHUNK)
    def _():
      ca, cb = gather_copies(k, b)
      ca.start()
      cb.start()

  def body(k, b):
    @pl.when(chunk_of(k + 1) < NCHUNK)
    def _():
      idx_copy(k + 1, 1 - b).wait()

    @pl.when((k >= 1) & (chunk_of(k - 1) < NCHUNK))
    def _():
      store_copy(k - 1, 1 - b).wait()

    issue_gather(k + 1, 1 - b)

    @pl.when(chunk_of(k) < NCHUNK)
    def _():
      ca, cb = gather_copies(k, b)
      ca.wait()
      cb.wait()

    issue_idx(k + 2, b)

    @pl.when(chunk_of(k) < NCHUNK)
    def _():
      sbuf = sb[b]
      tbuf = tb[b]

      def addrow(i, carry2):
        for j in range(_LANES):
          sl = pl.ds(j * 16, 16)
          sbuf[i, sl] = sbuf[i, sl] + tbuf[i, sl]
        return carry2

      lax.fori_loop(0, CH, addrow, 0)
      store_copy(k, b).start()

  issue_idx(0, 0)

  @pl.when(chunk_of(0) < NCHUNK)
  def _():
    idx_copy(0, 0).wait()

  issue_gather(0, 0)
  issue_idx(1, 1)

  def outer(j, carry):
    body(2 * j, 0)
    body(2 * j + 1, 1)
    return carry

  lax.fori_loop(0, CPW // 2, outer, 0)

  @pl.when(chunk_of(CPW - 1) < NCHUNK)
  def _():
    store_copy(CPW - 1, (CPW - 1) % 2).wait()


_sc_gather = functools.partial(
    pl.kernel,
    out_type=jax.ShapeDtypeStruct((E, H), jnp.float32),
    mesh=_sc_mesh(),
    scratch_types=[
        pltpu.VMEM((2, CH), jnp.int32),
        pltpu.VMEM((2, CH), jnp.int32),
        pltpu.VMEM((CH, H), jnp.float32),
        pltpu.VMEM((CH, H), jnp.float32),
        pltpu.VMEM((CH, H), jnp.float32),
        pltpu.VMEM((CH, H), jnp.float32),
        pltpu.SemaphoreType.DMA,
        pltpu.SemaphoreType.DMA,
        pltpu.SemaphoreType.DMA,
        pltpu.SemaphoreType.DMA,
        pltpu.SemaphoreType.DMA,
        pltpu.SemaphoreType.DMA,
    ],
)(_sc_gather_body)


def _sc_scatter_body(gm_hbm, rc_hbm, out_hbm,
                     rc0, rc1, rc2, vb0, vb1, vb2, acc,
                     seml0, seml1, seml2, sema0, sema1, sema2, semz):
  cid = lax.axis_index("c")
  sid = lax.axis_index("s")
  wid = sid * NCORE + cid
  rc = (rc0, rc1, rc2)
  vb = (vb0, vb1, vb2)
  seml = (seml0, seml1, seml2)
  sema = (sema0, sema1, sema2)

  def chunk_of(k):
    return wid + k * NW

  def load_copies(k, b):
    c = chunk_of(k)
    return (pltpu.make_async_copy(rc_hbm.at[c], rc[b], seml[b]),
            pltpu.make_async_copy(gm_hbm.at[pl.ds(c * CH, CH)], vb[b],
                                  seml[b]))

  def add_copy(k, b):
    return pltpu.make_async_copy(vb[b], acc.at[rc[b].at[1]], sema[b])

  def issue_load(k, b):
    @pl.when(chunk_of(k) < NCHUNK)
    def _():
      ca, cb = load_copies(k, b)
      ca.start()
      cb.start()

  issue_load(0, 0)
  issue_load(1, 1)

  def zrow(i, carry):
    for j in range(_LANES):
      vb2[i, pl.ds(j * 16, 16)] = jnp.zeros((16,), jnp.float32)
    return carry

  lax.fori_loop(0, ZC, zrow, 0)

  def zchunk(k, carry):
    z = sid + k * NSUB

    @pl.when(z < NZCHUNK)
    def _():
      pltpu.sync_copy(vb2.at[pl.ds(0, ZC)], acc.at[pl.ds(z * ZC, ZC)])

    return carry

  lax.fori_loop(0, ZPW, zchunk, 0)
  plsc.subcore_barrier()

  def step(k, b):
    @pl.when(chunk_of(k) < NCHUNK)
    def _():
      ca, cb = load_copies(k, b)
      ca.wait()
      cb.wait()
      add_copy(k, b).start(add=True)

    @pl.when((k >= 1) & (chunk_of(k - 1) < NCHUNK))
    def _():
      add_copy(k - 1, (b - 1) % 3).wait()

    issue_load(k + 2, (b + 2) % 3)

  def outer(j, carry):
    for b in (0, 1, 2):
      step(3 * j + b, b)
    return carry

  lax.fori_loop(0, (CPW - 1) // 3, outer, 0)
  step(CPW - 1, (CPW - 1) % 3)

  @pl.when(chunk_of(CPW - 1) < NCHUNK)
  def _():
    add_copy(CPW - 1, (CPW - 1) % 3).wait()

  plsc.subcore_barrier()

  def ochunk(k, carry):
    z = sid + k * NSUB

    @pl.when(z < NZCHUNK)
    def _():
      pltpu.async_copy(acc.at[pl.ds(z * ZC, ZC)],
                       out_hbm.at[cid, pl.ds(z * ZC, ZC)], semz)

    return carry

  lax.fori_loop(0, ZPW, ochunk, 0)

  def odrain(k, carry):
    z = sid + k * NSUB

    @pl.when(z < NZCHUNK)
    def _():
      pltpu.make_async_copy(acc.at[pl.ds(z * ZC, ZC)],
                            out_hbm.at[cid, pl.ds(z * ZC, ZC)], semz).wait()

    return carry

  lax.fori_loop(0, ZPW, odrain, 0)


_sc_scatter = functools.partial(
    pl.kernel,
    out_type=jax.ShapeDtypeStruct((NCORE, N, H), jnp.float32),
    mesh=_sc_mesh(),
    scratch_types=[
        pltpu.VMEM((2, CH), jnp.int32),
        pltpu.VMEM((2, CH), jnp.int32),
        pltpu.VMEM((2, CH), jnp.int32),
        pltpu.VMEM((CH, H), jnp.float32),
        pltpu.VMEM((CH, H), jnp.float32),
        pltpu.VMEM((CH, H), jnp.float32),
        pltpu.VMEM_SHARED((N, H), jnp.float32),
        pltpu.SemaphoreType.DMA,
        pltpu.SemaphoreType.DMA,
        pltpu.SemaphoreType.DMA,
        pltpu.SemaphoreType.DMA,
        pltpu.SemaphoreType.DMA,
        pltpu.SemaphoreType.DMA,
        pltpu.SemaphoreType.DMA,
    ],
)(_sc_scatter_body)


def _node_first_body(x_ref, wn_ref, bn_ref, wst_ref, s_ref, t_ref):
  h = jnp.dot(x_ref[...], wn_ref[...], preferred_element_type=jnp.float32)
  h = jnp.maximum(h + bn_ref[...], 0.0)
  st = jnp.dot(h, wst_ref[...], preferred_element_type=jnp.float32)
  s_ref[...] = st[:, :H]
  t_ref[...] = st[:, H:]


def _node_first(x, wn, bn, wst):
  return pl.pallas_call(
      _node_first_body,
      grid=(N // NB,),
      in_specs=[
          pl.BlockSpec((NB, H), lambda i: (i, 0)),
          pl.BlockSpec((H, H), lambda i: (0, 0)),
          pl.BlockSpec((1, H), lambda i: (0, 0)),
          pl.BlockSpec((H, 2 * H), lambda i: (0, 0)),
      ],
      out_specs=[pl.BlockSpec((NB, H), lambda i: (i, 0)),
                 pl.BlockSpec((NB, H), lambda i: (i, 0))],
      out_shape=[jax.ShapeDtypeStruct((N, H), jnp.float32),
                 jax.ShapeDtypeStruct((N, H), jnp.float32)],
  )(x, wn, bn, wst)


def _node_mid_body(p0_ref, p1_ref, wst_ref, s_ref, t_ref):
  h = jnp.maximum(p0_ref[...] + p1_ref[...], 0.0)
  st = jnp.dot(h, wst_ref[...], preferred_element_type=jnp.float32)
  s_ref[...] = st[:, :H]
  t_ref[...] = st[:, H:]


def _node_mid(p0, p1, wst):
  return pl.pallas_call(
      _node_mid_body,
      grid=(N // NB,),
      in_specs=[
          pl.BlockSpec((NB, H), lambda i: (i, 0)),
          pl.BlockSpec((NB, H), lambda i: (i, 0)),
          pl.BlockSpec((H, 2 * H), lambda i: (0, 0)),
      ],
      out_specs=[pl.BlockSpec((NB, H), lambda i: (i, 0)),
                 pl.BlockSpec((NB, H), lambda i: (i, 0))],
      out_shape=[jax.ShapeDtypeStruct((N, H), jnp.float32),
                 jax.ShapeDtypeStruct((N, H), jnp.float32)],
  )(p0, p1, wst)


def _node_head_body(p0_ref, p1_ref, wh_ref, bh_ref, sem_ref, num_ref, ver_ref):
  h = jnp.maximum(p0_ref[...] + p1_ref[...], 0.0)
  o = jnp.dot(h, wh_ref[...], preferred_element_type=jnp.float32) + bh_ref[...]
  sem_ref[...] = o[:, :32]
  num_ref[...] = o[:, 32:37]
  ver_ref[...] = o[:, 37:38]


def _node_head(p0, p1, wh, bh):
  return pl.pallas_call(
      _node_head_body,
      grid=(N // NB,),
      in_specs=[
          pl.BlockSpec((NB, H), lambda i: (i, 0)),
          pl.BlockSpec((NB, H), lambda i: (i, 0)),
          pl.BlockSpec((H, H), lambda i: (0, 0)),
          pl.BlockSpec((1, H), lambda i: (0, 0)),
      ],
      out_specs=[pl.BlockSpec((NB, 32), lambda i: (i, 0)),
                 pl.BlockSpec((NB, 5), lambda i: (i, 0)),
                 pl.BlockSpec((NB, 1), lambda i: (i, 0))],
      out_shape=[jax.ShapeDtypeStruct((N, 32), jnp.float32),
                 jax.ShapeDtypeStruct((N, 5), jnp.float32),
                 jax.ShapeDtypeStruct((N, 1), jnp.float32)],
  )(p0, p1, wh, bh)


def _edge_body(ea_ref, raw_ref, g_ref, we_ref, be_ref, a_ref, mb1_ref,
               w2_ref, mb2_ref, awm_ref, scal_ref, gm_ref):
  e_wide = jnp.dot(ea_ref[...].astype(jnp.bfloat16), we_ref[...],
                   preferred_element_type=jnp.float32)
  e = jnp.maximum(e_wide + be_ref[...], 0.0).reshape(EB, H)
  raw_p = raw_ref[0]
  rr = lax.broadcasted_iota(jnp.int32, (EB, EB // 128), 0) // 128
  cc = lax.broadcasted_iota(jnp.int32, (EB, EB // 128), 1)
  pick = jnp.where(rr == cc, 1.0, 0.0).astype(jnp.float32)
  q = jnp.dot(pick, raw_p, preferred_element_type=jnp.float32)
  lane = lax.broadcasted_iota(jnp.int32, (EB, 128), 1)
  ridx = lax.broadcasted_iota(jnp.int32, (EB, 128), 0) % 128
  raw = jnp.sum(jnp.where(lane == ridx, q, 0.0), axis=1, keepdims=True)
  eb = jnp.dot(e.astype(jnp.bfloat16), a_ref[...],
               preferred_element_type=jnp.float32) + mb1_ref[...]
  pre = jnp.maximum(g_ref[...] + eb, 0.0)
  msg = jnp.dot(pre.astype(jnp.bfloat16), w2_ref[...],
                preferred_element_type=jnp.float32) + mb2_ref[...]
  inv_t = scal_ref[0]
  awc = scal_ref[1]
  ab = scal_ref[2]
  conf = jax.nn.sigmoid(raw * inv_t)
  logits = jnp.dot(msg, awm_ref[...],
                   preferred_element_type=jnp.float32) + conf * awc + ab
  gm_ref[...] = msg * jax.nn.sigmoid(logits)


def _edge(ea, raw, g, we, be, a, mb1, w2, mb2, awm, scal):
  return pl.pallas_call(
      _edge_body,
      grid=(E // EB,),
      in_specs=[
          pl.BlockSpec((EB // 8, 128), lambda i: (i, 0)),
          pl.BlockSpec((1, EB // 128, 128), lambda i: (i, 0, 0)),
          pl.BlockSpec((EB, H), lambda i: (i, 0)),
          pl.BlockSpec((H, 8 * H), lambda i: (0, 0)),
          pl.BlockSpec((1, 8 * H), lambda i: (0, 0)),
          pl.BlockSpec((H, H), lambda i: (0, 0)),
          pl.BlockSpec((1, H), lambda i: (0, 0)),
          pl.BlockSpec((H, H), lambda i: (0, 0)),
          pl.BlockSpec((1, H), lambda i: (0, 0)),
          pl.BlockSpec((H, 1), lambda i: (0, 0)),
          pl.BlockSpec(memory_space=pltpu.SMEM),
      ],
      out_specs=pl.BlockSpec((EB, H), lambda i: (i, 0)),
      out_shape=jax.ShapeDtypeStruct((E, H), jnp.float32),
  )(ea, raw, g, we, be, a, mb1, w2, mb2, awm, scal)


def kernel(x, edge_index, edge_attr, raw_vlm_confidence,
           node_proj_w, node_proj_b, edge_proj_w, edge_proj_b, temperature,
           c1_mw1, c1_mb1, c1_mw2, c1_mb2, c1_aw, c1_ab,
           c2_mw1, c2_mb1, c2_mw2, c2_mb2, c2_aw, c2_ab,
           sem_w, sem_b, num_w, num_b, ver_w, ver_b):
  rc = jnp.stack([edge_index[0].reshape(NCHUNK, CH),
                  edge_index[1].reshape(NCHUNK, CH)], axis=1)
  eap = edge_attr.reshape(E // 8, 128)
  rawp = raw_vlm_confidence.reshape(E // EB, EB // 128, 128)

  wst1 = jnp.concatenate([c1_mw1[:H], c1_mw1[H:2 * H]], axis=1)
  wst2 = jnp.concatenate([c2_mw1[:H], c2_mw1[H:2 * H]], axis=1)
  a1 = c1_mw1[2 * H:]
  a2 = c2_mw1[2 * H:]
  scal1 = jnp.stack([1.0 / temperature[0], c1_aw[H, 0], c1_ab[0]])
  scal2 = jnp.stack([1.0 / temperature[0], c2_aw[H, 0], c2_ab[0]])

  we_big = jnp.zeros((H, 8 * H), jnp.float32)
  be_big = jnp.zeros((1, 8 * H), jnp.float32)
  for j in range(8):
    we_big = we_big.at[D_EDGE * j:D_EDGE * (j + 1), H * j:H * (j + 1)].set(
        edge_proj_w)
    be_big = be_big.at[:, H * j:H * (j + 1)].set(edge_proj_b[None, :])
  we_big = we_big.astype(jnp.bfloat16)

  wh = jnp.concatenate(
      [sem_w, num_w, ver_w, jnp.zeros((H, H - 38), jnp.float32)], axis=1)
  bh = jnp.concatenate(
      [sem_b, num_b, ver_b, jnp.zeros((H - 38,), jnp.float32)]).reshape(1, H)

  s1, t1 = _node_first(x, node_proj_w, node_proj_b.reshape(1, H), wst1)
  g1 = _sc_gather(s1, t1, rc)
  gm1 = _edge(eap, rawp, g1, we_big, be_big, a1.astype(jnp.bfloat16),
              c1_mb1.reshape(1, H), c1_mw2.astype(jnp.bfloat16),
              c1_mb2.reshape(1, H), c1_aw[:H], scal1)
  p1 = _sc_scatter(gm1, rc)

  s2, t2 = _node_mid(p1[0], p1[1], wst2)
  g2 = _sc_gather(s2, t2, rc)
  gm2 = _edge(eap, rawp, g2, we_big, be_big, a2.astype(jnp.bfloat16),
              c2_mb1.reshape(1, H), c2_mw2.astype(jnp.bfloat16),
              c2_mb2.reshape(1, H), c2_aw[:H], scal2)
  p2 = _sc_scatter(gm2, rc)

  sem, num, ver = _node_head(p2[0], p2[1], wh, bh)
  return (sem, num, ver)


# speedup vs baseline: 1.0115x; 1.0115x over previous
import functools

import jax
import jax.numpy as jnp
from jax import lax
from jax.experimental import pallas as pl
from jax.experimental.pallas import tpu as pltpu
from jax.experimental.pallas import tpu_sc as plsc

N = 10000
E = 160000
H = 128
D_EDGE = 16

NCORE = 2
NSUB = 16
NW = NCORE * NSUB

CH = 128
NCHUNK = E // CH
CPW = -(-NCHUNK // NW)
ZC = 80
NZCHUNK = N // ZC
ZPW = -(-NZCHUNK // NSUB)

NB = 1000
EB = 3200

_LANES = H // 16

_sc_mesh = lambda: plsc.VectorSubcoreMesh(
    core_axis_name="c", subcore_axis_name="s", num_cores=NCORE, num_subcores=NSUB)


def _sc_gather_body(s_hbm, t_hbm, rc_hbm, g_hbm,
                    rc0, rc1, sb0, tb0, sb1, tb1,
                    semi0, semi1, semg0, semg1, semo0, semo1):
  wid = lax.axis_index("s") * NCORE + lax.axis_index("c")
  rc = (rc0, rc1)
  sb = (sb0, sb1)
  tb = (tb0, tb1)
  semi = (semi0, semi1)
  semg = (semg0, semg1)
  semo = (semo0, semo1)

  def chunk_of(k):
    return wid + k * NW

  def idx_copy(k, b):
    return pltpu.make_async_copy(rc_hbm.at[chunk_of(k)], rc[b], semi[b])

  def gather_copies(k, b):
    return (pltpu.make_async_copy(s_hbm.at[rc[b].at[0]], sb[b], semg[b]),
            pltpu.make_async_copy(t_hbm.at[rc[b].at[1]], tb[b], semg[b]))

  def store_copy(k, b):
    return pltpu.make_async_copy(sb[b], g_hbm.at[pl.ds(chunk_of(k) * CH, CH)],
                                 semo[b])

  def issue_idx(k, b):
    @pl.when(chunk_of(k) < NCHUNK)
    def _():
      idx_copy(k, b).start()

  def issue_gather(k, b):
    @pl.when(chunk_of(k) < NCHUNK)
    def _():
      ca, cb = gather_copies(k, b)
      ca.start()
      cb.start()

  def body(k, b):
    @pl.when(chunk_of(k + 1) < NCHUNK)
    def _():
      idx_copy(k + 1, 1 - b).wait()

    @pl.when((k >= 1) & (chunk_of(k - 1) < NCHUNK))
    def _():
      store_copy(k - 1, 1 - b).wait()

    issue_gather(k + 1, 1 - b)

    @pl.when(chunk_of(k) < NCHUNK)
    def _():
      ca, cb = gather_copies(k, b)
      ca.wait()
      cb.wait()

    issue_idx(k + 2, b)

    @pl.when(chunk_of(k) < NCHUNK)
    def _():
      sbuf = sb[b]
      tbuf = tb[b]

      def addrow(i, carry2):
        for j in range(_LANES):
          sl = pl.ds(j * 16, 16)
          sbuf[i, sl] = sbuf[i, sl] + tbuf[i, sl]
        return carry2

      lax.fori_loop(0, CH, addrow, 0)
      store_copy(k, b).start()

  issue_idx(0, 0)

  @pl.when(chunk_of(0) < NCHUNK)
  def _():
    idx_copy(0, 0).wait()

  issue_gather(0, 0)
  issue_idx(1, 1)

  def outer(j, carry):
    body(2 * j, 0)
    body(2 * j + 1, 1)
    return carry

  lax.fori_loop(0, CPW // 2, outer, 0)

  @pl.when(chunk_of(CPW - 1) < NCHUNK)
  def _():
    store_copy(CPW - 1, (CPW - 1) % 2).wait()


_sc_gather = functools.partial(
    pl.kernel,
    out_type=jax.ShapeDtypeStruct((E, H), jnp.float32),
    mesh=_sc_mesh(),
    scratch_types=[
        pltpu.VMEM((2, CH), jnp.int32),
        pltpu.VMEM((2, CH), jnp.int32),
        pltpu.VMEM((CH, H), jnp.float32),
        pltpu.VMEM((CH, H), jnp.float32),
        pltpu.VMEM((CH, H), jnp.float32),
        pltpu.VMEM((CH, H), jnp.float32),
        pltpu.SemaphoreType.DMA,
        pltpu.SemaphoreType.DMA,
        pltpu.SemaphoreType.DMA,
        pltpu.SemaphoreType.DMA,
        pltpu.SemaphoreType.DMA,
        pltpu.SemaphoreType.DMA,
    ],
)(_sc_gather_body)


def _sc_scatter_body(gm_hbm, rc_hbm, out_hbm,
                     rc0, rc1, rc2, vb0, vb1, vb2, acc,
                     seml0, seml1, seml2, sema0, sema1, sema2, semz):
  cid = lax.axis_index("c")
  sid = lax.axis_index("s")
  wid = sid * NCORE + cid
  rc = (rc0, rc1, rc2)
  vb = (vb0, vb1, vb2)
  seml = (seml0, seml1, seml2)
  sema = (sema0, sema1, sema2)

  def chunk_of(k):
    return wid + k * NW

  def load_copies(k, b):
    c = chunk_of(k)
    return (pltpu.make_async_copy(rc_hbm.at[c], rc[b], seml[b]),
            pltpu.make_async_copy(gm_hbm.at[pl.ds(c * CH, CH)], vb[b],
                                  seml[b]))

  def add_copy(k, b):
    return pltpu.make_async_copy(vb[b], acc.at[rc[b].at[1]], sema[b])

  def issue_load(k, b):
    @pl.when(chunk_of(k) < NCHUNK)
    def _():
      ca, cb = load_copies(k, b)
      ca.start()
      cb.start()

  issue_load(0, 0)
  issue_load(1, 1)

  def zrow(i, carry):
    for j in range(_LANES):
      vb2[i, pl.ds(j * 16, 16)] = jnp.zeros((16,), jnp.float32)
    return carry

  lax.fori_loop(0, ZC, zrow, 0)

  def zchunk(k, carry):
    z = sid + k * NSUB

    @pl.when(z < NZCHUNK)
    def _():
      pltpu.sync_copy(vb2.at[pl.ds(0, ZC)], acc.at[pl.ds(z * ZC, ZC)])

    return carry

  lax.fori_loop(0, ZPW, zchunk, 0)
  plsc.subcore_barrier()

  def step(k, b):
    @pl.when(chunk_of(k) < NCHUNK)
    def _():
      ca, cb = load_copies(k, b)
      ca.wait()
      cb.wait()
      add_copy(k, b).start(add=True)

    @pl.when((k >= 1) & (chunk_of(k - 1) < NCHUNK))
    def _():
      add_copy(k - 1, (b - 1) % 3).wait()

    issue_load(k + 2, (b + 2) % 3)

  def outer(j, carry):
    for b in (0, 1, 2):
      step(3 * j + b, b)
    return carry

  lax.fori_loop(0, (CPW - 1) // 3, outer, 0)
  step(CPW - 1, (CPW - 1) % 3)

  @pl.when(chunk_of(CPW - 1) < NCHUNK)
  def _():
    add_copy(CPW - 1, (CPW - 1) % 3).wait()

  plsc.subcore_barrier()

  def ochunk(k, carry):
    z = sid + k * NSUB

    @pl.when(z < NZCHUNK)
    def _():
      pltpu.async_copy(acc.at[pl.ds(z * ZC, ZC)],
                       out_hbm.at[cid, pl.ds(z * ZC, ZC)], semz)

    return carry

  lax.fori_loop(0, ZPW, ochunk, 0)

  def odrain(k, carry):
    z = sid + k * NSUB

    @pl.when(z < NZCHUNK)
    def _():
      pltpu.make_async_copy(acc.at[pl.ds(z * ZC, ZC)],
                            out_hbm.at[cid, pl.ds(z * ZC, ZC)], semz).wait()

    return carry

  lax.fori_loop(0, ZPW, odrain, 0)


_sc_scatter = functools.partial(
    pl.kernel,
    out_type=jax.ShapeDtypeStruct((NCORE, N, H), jnp.float32),
    mesh=_sc_mesh(),
    scratch_types=[
        pltpu.VMEM((2, CH), jnp.int32),
        pltpu.VMEM((2, CH), jnp.int32),
        pltpu.VMEM((2, CH), jnp.int32),
        pltpu.VMEM((CH, H), jnp.float32),
        pltpu.VMEM((CH, H), jnp.float32),
        pltpu.VMEM((CH, H), jnp.float32),
        pltpu.VMEM_SHARED((N, H), jnp.float32),
        pltpu.SemaphoreType.DMA,
        pltpu.SemaphoreType.DMA,
        pltpu.SemaphoreType.DMA,
        pltpu.SemaphoreType.DMA,
        pltpu.SemaphoreType.DMA,
        pltpu.SemaphoreType.DMA,
        pltpu.SemaphoreType.DMA,
    ],
)(_sc_scatter_body)


def _node_first_body(x_ref, wn_ref, bn_ref, wst_ref, s_ref, t_ref):
  h = jnp.dot(x_ref[...], wn_ref[...], preferred_element_type=jnp.float32)
  h = jnp.maximum(h + bn_ref[...], 0.0)
  st = jnp.dot(h, wst_ref[...], preferred_element_type=jnp.float32)
  s_ref[...] = st[:, :H]
  t_ref[...] = st[:, H:]


def _node_first(x, wn, bn, wst):
  return pl.pallas_call(
      _node_first_body,
      grid=(N // NB,),
      in_specs=[
          pl.BlockSpec((NB, H), lambda i: (i, 0)),
          pl.BlockSpec((H, H), lambda i: (0, 0)),
          pl.BlockSpec((1, H), lambda i: (0, 0)),
          pl.BlockSpec((H, 2 * H), lambda i: (0, 0)),
      ],
      out_specs=[pl.BlockSpec((NB, H), lambda i: (i, 0)),
                 pl.BlockSpec((NB, H), lambda i: (i, 0))],
      out_shape=[jax.ShapeDtypeStruct((N, H), jnp.float32),
                 jax.ShapeDtypeStruct((N, H), jnp.float32)],
  )(x, wn, bn, wst)


def _node_mid_body(p0_ref, p1_ref, wst_ref, s_ref, t_ref):
  h = jnp.maximum(p0_ref[...] + p1_ref[...], 0.0)
  st = jnp.dot(h, wst_ref[...], preferred_element_type=jnp.float32)
  s_ref[...] = st[:, :H]
  t_ref[...] = st[:, H:]


def _node_mid(p0, p1, wst):
  return pl.pallas_call(
      _node_mid_body,
      grid=(N // NB,),
      in_specs=[
          pl.BlockSpec((NB, H), lambda i: (i, 0)),
          pl.BlockSpec((NB, H), lambda i: (i, 0)),
          pl.BlockSpec((H, 2 * H), lambda i: (0, 0)),
      ],
      out_specs=[pl.BlockSpec((NB, H), lambda i: (i, 0)),
                 pl.BlockSpec((NB, H), lambda i: (i, 0))],
      out_shape=[jax.ShapeDtypeStruct((N, H), jnp.float32),
                 jax.ShapeDtypeStruct((N, H), jnp.float32)],
  )(p0, p1, wst)


def _node_head_body(p0_ref, p1_ref, wh_ref, bh_ref, sem_ref, num_ref, ver_ref):
  h = jnp.maximum(p0_ref[...] + p1_ref[...], 0.0)
  o = jnp.dot(h, wh_ref[...], preferred_element_type=jnp.float32) + bh_ref[...]
  sem_ref[...] = o[:, :32]
  num_ref[...] = o[:, 32:37]
  ver_ref[...] = o[:, 37:38]


def _node_head(p0, p1, wh, bh):
  return pl.pallas_call(
      _node_head_body,
      grid=(N // NB,),
      in_specs=[
          pl.BlockSpec((NB, H), lambda i: (i, 0)),
          pl.BlockSpec((NB, H), lambda i: (i, 0)),
          pl.BlockSpec((H, H), lambda i: (0, 0)),
          pl.BlockSpec((1, H), lambda i: (0, 0)),
      ],
      out_specs=[pl.BlockSpec((NB, 32), lambda i: (i, 0)),
                 pl.BlockSpec((NB, 5), lambda i: (i, 0)),
                 pl.BlockSpec((NB, 1), lambda i: (i, 0))],
      out_shape=[jax.ShapeDtypeStruct((N, 32), jnp.float32),
                 jax.ShapeDtypeStruct((N, 5), jnp.float32),
                 jax.ShapeDtypeStruct((N, 1), jnp.float32)],
  )(p0, p1, wh, bh)


def _edge_body(ea_ref, raw_ref, g_ref, we_ref, be_ref, a_ref, mb1_ref,
               w2_ref, mb2_ref, awm_ref, scal_ref, gm_ref):
  e_wide = jnp.dot(ea_ref[...].astype(jnp.bfloat16), we_ref[...],
                   preferred_element_type=jnp.float32)
  e = jnp.maximum(e_wide + be_ref[...], 0.0).astype(jnp.bfloat16).reshape(EB, H)
  inv_t = scal_ref[0]
  awc = scal_ref[1]
  ab = scal_ref[2]
  ct_p = jax.nn.sigmoid(raw_ref[0] * inv_t) * awc + ab
  rr = lax.broadcasted_iota(jnp.int32, (EB, EB // 128), 0) // 128
  cc = lax.broadcasted_iota(jnp.int32, (EB, EB // 128), 1)
  pick = jnp.where(rr == cc, 1.0, 0.0).astype(jnp.float32)
  q = jnp.dot(pick, ct_p, preferred_element_type=jnp.float32)
  lane = lax.broadcasted_iota(jnp.int32, (EB, 128), 1)
  ridx = lax.broadcasted_iota(jnp.int32, (EB, 128), 0) % 128
  ct = jnp.sum(jnp.where(lane == ridx, q, 0.0), axis=1, keepdims=True)
  eb = jnp.dot(e, a_ref[...],
               preferred_element_type=jnp.float32) + mb1_ref[...]
  pre = jnp.maximum(g_ref[...] + eb, 0.0)
  msg = jnp.dot(pre.astype(jnp.bfloat16), w2_ref[...],
                preferred_element_type=jnp.float32) + mb2_ref[...]
  logits = jnp.dot(msg, awm_ref[...],
                   preferred_element_type=jnp.float32) + ct
  gm_ref[...] = msg * jax.nn.sigmoid(logits)


def _edge(ea, raw, g, we, be, a, mb1, w2, mb2, awm, scal):
  return pl.pallas_call(
      _edge_body,
      grid=(E // EB,),
      in_specs=[
          pl.BlockSpec((EB // 8, 128), lambda i: (i, 0)),
          pl.BlockSpec((1, EB // 128, 128), lambda i: (i, 0, 0)),
          pl.BlockSpec((EB, H), lambda i: (i, 0)),
          pl.BlockSpec((H, 8 * H), lambda i: (0, 0)),
          pl.BlockSpec((1, 8 * H), lambda i: (0, 0)),
          pl.BlockSpec((H, H), lambda i: (0, 0)),
          pl.BlockSpec((1, H), lambda i: (0, 0)),
          pl.BlockSpec((H, H), lambda i: (0, 0)),
          pl.BlockSpec((1, H), lambda i: (0, 0)),
          pl.BlockSpec((H, 1), lambda i: (0, 0)),
          pl.BlockSpec(memory_space=pltpu.SMEM),
      ],
      out_specs=pl.BlockSpec((EB, H), lambda i: (i, 0)),
      out_shape=jax.ShapeDtypeStruct((E, H), jnp.float32),
  )(ea, raw, g, we, be, a, mb1, w2, mb2, awm, scal)


def kernel(x, edge_index, edge_attr, raw_vlm_confidence,
           node_proj_w, node_proj_b, edge_proj_w, edge_proj_b, temperature,
           c1_mw1, c1_mb1, c1_mw2, c1_mb2, c1_aw, c1_ab,
           c2_mw1, c2_mb1, c2_mw2, c2_mb2, c2_aw, c2_ab,
           sem_w, sem_b, num_w, num_b, ver_w, ver_b):
  rc = jnp.stack([edge_index[0].reshape(NCHUNK, CH),
                  edge_index[1].reshape(NCHUNK, CH)], axis=1)
  eap = edge_attr.reshape(E // 8, 128)
  rawp = raw_vlm_confidence.reshape(E // EB, EB // 128, 128)

  wst1 = jnp.concatenate([c1_mw1[:H], c1_mw1[H:2 * H]], axis=1)
  wst2 = jnp.concatenate([c2_mw1[:H], c2_mw1[H:2 * H]], axis=1)
  a1 = c1_mw1[2 * H:]
  a2 = c2_mw1[2 * H:]
  scal1 = jnp.stack([1.0 / temperature[0], c1_aw[H, 0], c1_ab[0]])
  scal2 = jnp.stack([1.0 / temperature[0], c2_aw[H, 0], c2_ab[0]])

  we_big = jnp.zeros((H, 8 * H), jnp.float32)
  be_big = jnp.zeros((1, 8 * H), jnp.float32)
  for j in range(8):
    we_big = we_big.at[D_EDGE * j:D_EDGE * (j + 1), H * j:H * (j + 1)].set(
        edge_proj_w)
    be_big = be_big.at[:, H * j:H * (j + 1)].set(edge_proj_b[None, :])
  we_big = we_big.astype(jnp.bfloat16)

  wh = jnp.concatenate(
      [sem_w, num_w, ver_w, jnp.zeros((H, H - 38), jnp.float32)], axis=1)
  bh = jnp.concatenate(
      [sem_b, num_b, ver_b, jnp.zeros((H - 38,), jnp.float32)]).reshape(1, H)

  s1, t1 = _node_first(x, node_proj_w, node_proj_b.reshape(1, H), wst1)
  g1 = _sc_gather(s1, t1, rc)
  gm1 = _edge(eap, rawp, g1, we_big, be_big, a1.astype(jnp.bfloat16),
              c1_mb1.reshape(1, H), c1_mw2.astype(jnp.bfloat16),
              c1_mb2.reshape(1, H), c1_aw[:H], scal1)
  p1 = _sc_scatter(gm1, rc)

  s2, t2 = _node_mid(p1[0], p1[1], wst2)
  g2 = _sc_gather(s2, t2, rc)
  gm2 = _edge(eap, rawp, g2, we_big, be_big, a2.astype(jnp.bfloat16),
              c2_mb1.reshape(1, H), c2_mw2.astype(jnp.bfloat16),
              c2_mb2.reshape(1, H), c2_aw[:H], scal2)
  p2 = _sc_scatter(gm2, rc)

  sem, num, ver = _node_head(p2[0], p2[1], wh, bh)
  return (sem, num, ver)
